# baseline (speedup 1.0000x reference)
import functools

import jax
import jax.numpy as jnp
from jax.experimental import pallas as pl
from jax.experimental.pallas import tpu as pltpu

_NBUF = 4


def _elu(v):
    return jnp.where(v > 0, v, jnp.exp(jnp.minimum(v, 0.0)) - 1.0)


def _passes_kernel(x_ref, w1_ref, adj_hbm, w23_ref, c_ref, lw1_ref, lb1_ref,
                   lw2_ref, lb2_ref, lw3_ref, lb3_ref,
                   mu_ref, lv_ref, z_ref, label_ref,
                   buf, xw1_s, g_s, z_s, sems, *, ni, bi, h2):
    i = pl.program_id(0)

    def row_of(c):
        return jnp.where(c < ni, c, c - ni) * bi

    def start_copy(c):
        slot = jax.lax.rem(c, _NBUF)
        pltpu.make_async_copy(
            adj_hbm.at[pl.ds(row_of(c), bi), :],
            buf.at[slot],
            sems.at[slot],
        ).start()

    @pl.when(i == 0)
    def _():
        for j in range(_NBUF - 1):
            start_copy(jnp.int32(j))
        xw1_s[...] = jnp.dot(x_ref[...], w1_ref[...],
                             preferred_element_type=jnp.float32)

    nxt = i + _NBUF - 1
    @pl.when(nxt < 2 * ni)
    def _():
        start_copy(nxt)

    slot = jax.lax.rem(i, _NBUF)
    pltpu.make_async_copy(
        adj_hbm.at[pl.ds(row_of(i), bi), :], buf.at[slot], sems.at[slot]
    ).wait()
    adj_blk = buf[slot]

    @pl.when(i < ni)
    def _():
        h = jnp.maximum(jnp.dot(adj_blk, xw1_s[...],
                                preferred_element_type=jnp.float32), 0.0)
        g_s[pl.ds(i * bi, bi), :] = jnp.dot(
            h, w23_ref[...], preferred_element_type=jnp.float32)

    @pl.when(i >= ni)
    def _():
        k = i - ni
        acc = jnp.dot(adj_blk, g_s[...], preferred_element_type=jnp.float32)
        mu = acc[:, :h2]
        mu_ref[...] = mu
        lv_ref[...] = acc[:, h2:]
        z = jnp.dot(mu, c_ref[...], preferred_element_type=jnp.float32)
        z_ref[...] = z
        z_s[pl.ds(k * bi, bi), :] = z
        h = _elu(jnp.dot(z, lw1_ref[...], preferred_element_type=jnp.float32)
                 + lb1_ref[...])
        h = _elu(jnp.dot(h, lw2_ref[...], preferred_element_type=jnp.float32)
                 + lb2_ref[...])
        label_ref[...] = (jnp.dot(h, lw3_ref[...],
                                  preferred_element_type=jnp.float32)
                          + lb3_ref[...])


def _decoder_kernel(z_ref, zall_ref, o_ref):
    o_ref[...] = jax.lax.dot_general(
        z_ref[...], zall_ref[...],
        dimension_numbers=(((1,), (1,)), ((), ())),
        preferred_element_type=jnp.float32)


def kernel(x, adj, W1, W2, W3, C, lw1, lb1, lw2, lb2, lw3, lb3):
    n, d_in = x.shape
    h1 = W1.shape[1]
    h2 = W2.shape[1]
    w23 = jnp.concatenate([W2, W3], axis=1)
    lb1r = lb1.reshape(1, -1)
    lb2r = lb2.reshape(1, -1)
    lb3r = lb3.reshape(1, -1)

    bi = 200 if n % 200 == 0 else n
    ni = n // bi

    p2_idx = lambda i: (jnp.maximum(i - ni, 0), 0)
    const = lambda a: pl.BlockSpec(a.shape, lambda i: (0,) * a.ndim)

    mu, logvar, z, label = pl.pallas_call(
        functools.partial(_passes_kernel, ni=ni, bi=bi, h2=h2),
        grid=(2 * ni,),
        in_specs=[
            const(x), const(W1),
            pl.BlockSpec(memory_space=pl.ANY),
            const(w23), const(C), const(lw1), const(lb1r), const(lw2),
            const(lb2r), const(lw3), const(lb3r),
        ],
        out_specs=[
            pl.BlockSpec((bi, h2), p2_idx),
            pl.BlockSpec((bi, h2), p2_idx),
            pl.BlockSpec((bi, h2), p2_idx),
            pl.BlockSpec((bi, d_in), p2_idx),
        ],
        out_shape=[
            jax.ShapeDtypeStruct((n, h2), jnp.float32),
            jax.ShapeDtypeStruct((n, h2), jnp.float32),
            jax.ShapeDtypeStruct((n, h2), jnp.float32),
            jax.ShapeDtypeStruct((n, d_in), jnp.float32),
        ],
        scratch_shapes=[
            pltpu.VMEM((_NBUF, bi, n), jnp.float32),
            pltpu.VMEM((n, h1), jnp.float32),
            pltpu.VMEM((n, 2 * h2), jnp.float32),
            pltpu.VMEM((n, h2), jnp.float32),
            pltpu.SemaphoreType.DMA((_NBUF,)),
        ],
    )(x, W1, adj, w23, C, lw1, lb1r, lw2, lb2r, lw3, lb3r)

    bd = 400 if n % 400 == 0 else n
    adj_rec = pl.pallas_call(
        _decoder_kernel,
        grid=(n // bd,),
        in_specs=[
            pl.BlockSpec((bd, h2), lambda i: (i, 0)),
            pl.BlockSpec((n, h2), lambda i: (0, 0)),
        ],
        out_specs=pl.BlockSpec((bd, n), lambda i: (i, 0)),
        out_shape=jax.ShapeDtypeStruct((n, n), jnp.float32),
    )(z, z)

    return (label, adj_rec, mu, logvar, mu, z)
